# Initial kernel scaffold; baseline (speedup 1.0000x reference)
import jax
import jax.numpy as jnp
from jax.experimental import pallas as pl


def kernel(x, candidates, labels, group_labels, W1, b1, Wm, bm, Etab, group_y):
    raise NotImplementedError("write your pallas kernel here")


# speedup vs baseline: 8.3783x; 8.3783x over previous
import functools

import jax
import jax.numpy as jnp
from jax import lax
from jax.experimental import pallas as pl
from jax.experimental.pallas import tpu as pltpu
from jax.experimental.pallas import tpu_sc as plsc

B, D, H, G, K, L, C = 1024, 5000, 1000, 64, 8, 4096, 32
GS = L // G
KC = K * GS
DP, HP = 5120, 1024
BB = 128
NB = B // BB
DK = 1280
ND = DP // DK


def _main_body(x_ref, w1_ref, b1_ref, wm_ref, bm_ref, et_ref, cnd_ref,
               lab_ref, gl_ref, z_ref, ridx_ref, nl_ref, s2_ref, acc_ref):
    ib = pl.program_id(0)
    idd = pl.program_id(1)

    @pl.when(idd == 0)
    def _():
        acc_ref[:] = jnp.zeros_like(acc_ref)

    acc_ref[:] += jnp.dot(x_ref[:], w1_ref[:], preferred_element_type=jnp.float32)

    @pl.when(idd == ND - 1)
    def _():
        h = jnp.maximum(acc_ref[:] + b1_ref[:], 0.0)
        glog = jnp.dot(h, wm_ref[:], preferred_element_type=jnp.float32) + bm_ref[:]
        y = 1.0 / (1.0 + jnp.exp(-glog))

        ji = lax.broadcasted_iota(jnp.int32, (BB, G), 1)
        work = y
        idx_cols = []
        for _ in range(K):
            m = jnp.max(work, axis=1, keepdims=True)
            sel = jnp.min(jnp.where(work == m, ji, G), axis=1, keepdims=True)
            idx_cols.append(sel)
            work = jnp.where(ji == sel, -1.0, work)

        ci = lax.broadcasted_iota(jnp.int32, (BB, KC), 1)
        s = jnp.bitwise_and(ci, GS - 1)
        kk = lax.shift_right_logical(ci, 6)
        grp = jnp.zeros((BB, KC), jnp.int32)
        for k in range(K):
            grp = jnp.where(kk == k, idx_cols[k], grp)
        cand = grp * GS + s

        bi = lax.broadcasted_iota(jnp.int32, (BB, 1), 0) + ib * BB
        ridx_ref[:] = bi * L + cand
        nl = jnp.zeros((BB, KC), jnp.bool_)
        for j in range(C):
            nl = jnp.logical_or(
                nl, jnp.logical_and(cand == cnd_ref[:, j:j + 1],
                                    lab_ref[:, j:j + 1] != 0))
        nl_ref[:] = nl.astype(jnp.float32)

        z_ref[:] = jnp.dot(y, et_ref[:], preferred_element_type=jnp.float32)

        t = gl_ref[:].astype(jnp.float32)
        part = jnp.sum(jnp.maximum(y, 0.0) - y * t
                       + jnp.log1p(jnp.exp(-jnp.abs(y))))
        prev = jnp.where(ib == 0, 0.0, s2_ref[0, 0])
        s2_ref[0, 0] = prev + part


_NW = 32
_IDX_ROWS = (B * KC) // 128
_RPW = _IDX_ROWS // _NW


@functools.partial(
    pl.kernel,
    out_type=jax.ShapeDtypeStruct((_IDX_ROWS, 128), jnp.float32),
    mesh=plsc.VectorSubcoreMesh(core_axis_name="c", subcore_axis_name="s"),
    scratch_types=[
        pltpu.VMEM((_RPW, 128), jnp.int32),
        pltpu.VMEM((_RPW, 128), jnp.float32),
        pltpu.SemaphoreType.DMA,
    ],
)
def _sc_gather(zflat_hbm, fidx_hbm, out_hbm, idx_v, out_v, sem):
    wid = lax.axis_index("s") * 2 + lax.axis_index("c")
    base = wid * _RPW
    pltpu.sync_copy(fidx_hbm.at[pl.ds(base, _RPW)], idx_v)
    cps = [pltpu.async_copy(zflat_hbm.at[idx_v.at[j]], out_v.at[j], sem)
           for j in range(_RPW)]
    for cp in cps:
        cp.wait()
    pltpu.sync_copy(out_v, out_hbm.at[pl.ds(base, _RPW)])


def _loss_body(lg_ref, nl_ref, s2_ref, out_ref):
    l = lg_ref[:]
    t = nl_ref[:]
    s1 = jnp.sum(jnp.maximum(l, 0.0) - l * t + jnp.log1p(jnp.exp(-jnp.abs(l))))
    out_ref[0, 0] = s1 / (B * KC) + s2_ref[0, 0] / (B * G)


def kernel(x, candidates, labels, group_labels, W1, b1, Wm, bm, Etab, group_y):
    xp = jnp.pad(x, ((0, 0), (0, DP - D)))
    w1p = jnp.pad(W1, ((0, DP - D), (0, HP - H)))
    b1p = jnp.pad(b1, (0, HP - H)).reshape(1, HP)
    wmp = jnp.pad(Wm, ((0, HP - H), (0, 0)))
    bmr = bm.reshape(1, G)
    etT = Etab.T

    z, ridx, nlab, s2 = pl.pallas_call(
        _main_body,
        grid=(NB, ND),
        in_specs=[
            pl.BlockSpec((BB, DK), lambda ib, idd: (ib, idd)),
            pl.BlockSpec((DK, HP), lambda ib, idd: (idd, 0)),
            pl.BlockSpec((1, HP), lambda ib, idd: (0, 0)),
            pl.BlockSpec((HP, G), lambda ib, idd: (0, 0)),
            pl.BlockSpec((1, G), lambda ib, idd: (0, 0)),
            pl.BlockSpec((G, L), lambda ib, idd: (0, 0)),
            pl.BlockSpec((BB, C), lambda ib, idd: (ib, 0)),
            pl.BlockSpec((BB, C), lambda ib, idd: (ib, 0)),
            pl.BlockSpec((BB, G), lambda ib, idd: (ib, 0)),
        ],
        out_specs=[
            pl.BlockSpec((BB, L), lambda ib, idd: (ib, 0)),
            pl.BlockSpec((BB, KC), lambda ib, idd: (ib, 0)),
            pl.BlockSpec((BB, KC), lambda ib, idd: (ib, 0)),
            pl.BlockSpec(memory_space=pltpu.SMEM),
        ],
        out_shape=[
            jax.ShapeDtypeStruct((B, L), jnp.float32),
            jax.ShapeDtypeStruct((B, KC), jnp.int32),
            jax.ShapeDtypeStruct((B, KC), jnp.float32),
            jax.ShapeDtypeStruct((1, 1), jnp.float32),
        ],
        scratch_shapes=[pltpu.VMEM((BB, HP), jnp.float32)],
    )(xp, w1p, b1p, wmp, bmr, etT, candidates, labels, group_labels)

    zflat = z.reshape(B * L)
    fidx = ridx.reshape(_IDX_ROWS, 128)
    logits = _sc_gather(zflat, fidx).reshape(B, KC)

    loss = pl.pallas_call(
        _loss_body,
        in_specs=[
            pl.BlockSpec((B, KC), lambda: (0, 0)),
            pl.BlockSpec((B, KC), lambda: (0, 0)),
            pl.BlockSpec(memory_space=pltpu.SMEM),
        ],
        out_specs=pl.BlockSpec(memory_space=pltpu.SMEM),
        out_shape=jax.ShapeDtypeStruct((1, 1), jnp.float32),
    )(logits, nlab, s2)

    return logits, loss.reshape(())


# speedup vs baseline: 10.4096x; 1.2425x over previous
import functools

import jax
import jax.numpy as jnp
from jax import lax
from jax.experimental import pallas as pl
from jax.experimental.pallas import tpu as pltpu
from jax.experimental.pallas import tpu_sc as plsc

B, D, H, G, K, L, C = 1024, 5000, 1000, 64, 8, 4096, 32
GS = L // G
KC = K * GS
BB = 128
NB = B // BB
DK = 1664
ND = 3
DT = D - ND * DK
NZ = L // 128
NQ = KC // 128


def _main_body(x_ref, xt_ref, w1_ref, w1t_ref, b1_ref, wm_ref, bm_ref,
               et_ref, cnd_ref, lab_ref, gl_ref,
               z_ref, fidx_ref, nl_ref, s2_ref, acc_ref):
    ib = pl.program_id(0)
    idd = pl.program_id(1)

    @pl.when(idd == 0)
    def _():
        acc_ref[:] = jnp.zeros_like(acc_ref)

    acc_ref[:] += jnp.dot(x_ref[:], w1_ref[:], preferred_element_type=jnp.float32)

    @pl.when(idd == ND - 1)
    def _():
        h = jnp.maximum(
            acc_ref[:]
            + jnp.dot(xt_ref[:], w1t_ref[:], preferred_element_type=jnp.float32)
            + b1_ref[:], 0.0)
        glog = jnp.dot(h, wm_ref[:], preferred_element_type=jnp.float32) + bm_ref[:]
        y = 1.0 / (1.0 + jnp.exp(-glog))

        ji = lax.broadcasted_iota(jnp.int32, (BB, G), 1)
        work = y
        idx_cols = []
        for _ in range(K):
            m = jnp.max(work, axis=1, keepdims=True)
            sel = jnp.min(jnp.where(work == m, ji, G), axis=1, keepdims=True)
            idx_cols.append(sel)
            work = jnp.where(ji == sel, -1.0, work)

        ci = lax.broadcasted_iota(jnp.int32, (BB, KC), 1)
        s = jnp.bitwise_and(ci, GS - 1)
        kk = lax.shift_right_logical(ci, 6)
        grp = jnp.zeros((BB, KC), jnp.int32)
        for k in range(K):
            grp = jnp.where(kk == k, idx_cols[k], grp)
        cand = grp * GS + s
        nl = jnp.zeros((BB, KC), jnp.bool_)
        for j in range(C):
            nl = jnp.logical_or(
                nl, jnp.logical_and(cand == cnd_ref[:, j:j + 1],
                                    lab_ref[:, j:j + 1] != 0))
        nl_ref[:] = nl.astype(jnp.float32)

        bi = lax.broadcasted_iota(jnp.int32, (BB, 1), 0) + ib * BB
        fx = (lax.shift_right_logical(cand, 7) * (B * 128)
              + bi * 128 + jnp.bitwise_and(cand, 127))
        for q in range(NQ):
            fidx_ref[q] = fx[:, q * 128:(q + 1) * 128]

        zz = jnp.dot(y, et_ref[:], preferred_element_type=jnp.float32)
        for j in range(NZ):
            z_ref[j] = zz[:, j * 128:(j + 1) * 128]

        t = gl_ref[:].astype(jnp.float32)
        part = jnp.sum(jnp.maximum(y, 0.0) - y * t
                       + jnp.log1p(jnp.exp(-jnp.abs(y))))
        prev = jnp.where(ib == 0, 0.0, s2_ref[0, 0])
        s2_ref[0, 0] = prev + part


_NW = 32
_IDX_ROWS = (B * KC) // 128
_RPW = _IDX_ROWS // _NW


@functools.partial(
    pl.kernel,
    out_type=jax.ShapeDtypeStruct((B, KC), jnp.float32),
    mesh=plsc.VectorSubcoreMesh(core_axis_name="c", subcore_axis_name="s"),
    scratch_types=[
        pltpu.VMEM((_RPW, 128), jnp.int32),
        pltpu.VMEM((_RPW, 128), jnp.float32),
        pltpu.SemaphoreType.DMA,
    ],
)
def _sc_gather(zflat_hbm, fidx_hbm, out_hbm, idx_v, out_v, sem):
    wid = lax.axis_index("s") * 2 + lax.axis_index("c")
    base = wid * _RPW
    q = wid // (B // 128)
    b0 = (wid % (B // 128)) * 128
    pltpu.sync_copy(fidx_hbm.at[pl.ds(base, _RPW)], idx_v)
    cps = [pltpu.async_copy(zflat_hbm.at[idx_v.at[j]], out_v.at[j], sem)
           for j in range(_RPW)]
    for cp in cps:
        cp.wait()
    pltpu.sync_copy(out_v, out_hbm.at[pl.ds(b0, 128), pl.ds(q * 128, 128)])


def _loss_body(lg_ref, nl_ref, s2_ref, out_ref):
    l = lg_ref[:]
    t = nl_ref[:]
    s1 = jnp.sum(jnp.maximum(l, 0.0) - l * t + jnp.log1p(jnp.exp(-jnp.abs(l))))
    out_ref[0, 0] = s1 / (B * KC) + s2_ref[0, 0] / (B * G)


def kernel(x, candidates, labels, group_labels, W1, b1, Wm, bm, Etab, group_y):
    x_tail = lax.slice(x, (0, ND * DK), (B, D))
    w1_tail = lax.slice(W1, (ND * DK, 0), (D, H))
    b1r = b1.reshape(1, H)
    bmr = bm.reshape(1, G)
    etT = Etab.T

    z3, fidx4, nlab, s2 = pl.pallas_call(
        _main_body,
        grid=(NB, ND),
        in_specs=[
            pl.BlockSpec((BB, DK), lambda ib, idd: (ib, idd)),
            pl.BlockSpec((BB, DT), lambda ib, idd: (ib, 0)),
            pl.BlockSpec((DK, H), lambda ib, idd: (idd, 0)),
            pl.BlockSpec((DT, H), lambda ib, idd: (0, 0)),
            pl.BlockSpec((1, H), lambda ib, idd: (0, 0)),
            pl.BlockSpec((H, G), lambda ib, idd: (0, 0)),
            pl.BlockSpec((1, G), lambda ib, idd: (0, 0)),
            pl.BlockSpec((G, L), lambda ib, idd: (0, 0)),
            pl.BlockSpec((BB, C), lambda ib, idd: (ib, 0)),
            pl.BlockSpec((BB, C), lambda ib, idd: (ib, 0)),
            pl.BlockSpec((BB, G), lambda ib, idd: (ib, 0)),
        ],
        out_specs=[
            pl.BlockSpec((NZ, BB, 128), lambda ib, idd: (0, ib, 0)),
            pl.BlockSpec((NQ, BB, 128), lambda ib, idd: (0, ib, 0)),
            pl.BlockSpec((BB, KC), lambda ib, idd: (ib, 0)),
            pl.BlockSpec(memory_space=pltpu.SMEM),
        ],
        out_shape=[
            jax.ShapeDtypeStruct((NZ, B, 128), jnp.float32),
            jax.ShapeDtypeStruct((NQ, B, 128), jnp.int32),
            jax.ShapeDtypeStruct((B, KC), jnp.float32),
            jax.ShapeDtypeStruct((1, 1), jnp.float32),
        ],
        scratch_shapes=[pltpu.VMEM((BB, H), jnp.float32)],
    )(x, x_tail, W1, w1_tail, b1r, Wm, bmr, etT, candidates, labels,
      group_labels)

    zflat = z3.reshape(NZ * B * 128)
    fidx = fidx4.reshape(_IDX_ROWS, 128)
    logits = _sc_gather(zflat, fidx)

    loss = pl.pallas_call(
        _loss_body,
        in_specs=[
            pl.BlockSpec((B, KC), lambda: (0, 0)),
            pl.BlockSpec((B, KC), lambda: (0, 0)),
            pl.BlockSpec(memory_space=pltpu.SMEM),
        ],
        out_specs=pl.BlockSpec(memory_space=pltpu.SMEM),
        out_shape=jax.ShapeDtypeStruct((1, 1), jnp.float32),
    )(logits, nlab, s2)

    return logits, loss.reshape(())


# speedup vs baseline: 12.5798x; 1.2085x over previous
import functools

import jax
import jax.numpy as jnp
from jax import lax
from jax.experimental import pallas as pl
from jax.experimental.pallas import tpu as pltpu
from jax.experimental.pallas import tpu_sc as plsc

B, D, H, G, K, L, C = 1024, 5000, 1000, 64, 8, 4096, 32
GS = L // G
KC = K * GS
BB = 128
NB = B // BB
DK = 1664
ND = 3
DT = D - ND * DK
NZ = L // 128
NQ = KC // 128


def _main_body(x_ref, xt_ref, w1_ref, w1t_ref, b1_ref, wm_ref, bm_ref,
               et_ref, cnd_ref, lab_ref, gl_ref,
               z_ref, fidx_ref, cfidx_ref, ks_ref, s2_ref, acc_ref):
    idd = pl.program_id(0)
    ib = pl.program_id(1)
    rows = pl.ds(ib * BB, BB)

    @pl.when(idd == 0)
    def _():
        acc_ref[rows, :] = jnp.zeros((BB, H), jnp.float32)

    acc_ref[rows, :] += jnp.dot(x_ref[:], w1_ref[:],
                                preferred_element_type=jnp.float32)

    @pl.when(idd == ND - 1)
    def _():
        h = jnp.maximum(
            acc_ref[rows, :]
            + jnp.dot(xt_ref[:], w1t_ref[:], preferred_element_type=jnp.float32)
            + b1_ref[:], 0.0)
        glog = jnp.dot(h, wm_ref[:], preferred_element_type=jnp.float32) + bm_ref[:]
        y = 1.0 / (1.0 + jnp.exp(-glog))

        ji = lax.broadcasted_iota(jnp.int32, (BB, G), 1)
        work = y
        idx_cols = []
        for _ in range(K):
            m = jnp.max(work, axis=1, keepdims=True)
            sel = jnp.min(jnp.where(work == m, ji, G), axis=1, keepdims=True)
            idx_cols.append(sel)
            work = jnp.where(ji == sel, -1.0, work)

        ci = lax.broadcasted_iota(jnp.int32, (BB, KC), 1)
        s = jnp.bitwise_and(ci, GS - 1)
        kk = lax.shift_right_logical(ci, 6)
        grp = jnp.zeros((BB, KC), jnp.int32)
        for k in range(K):
            grp = jnp.where(kk == k, idx_cols[k], grp)
        cand = grp * GS + s

        bi = lax.broadcasted_iota(jnp.int32, (BB, 1), 0) + ib * BB

        fx = (lax.shift_right_logical(cand, 7) * (B * 128)
              + bi * 128 + jnp.bitwise_and(cand, 127))
        for q in range(NQ):
            fidx_ref[q] = fx[:, q * 128:(q + 1) * 128]

        cval = cnd_ref[:]
        cfidx_ref[:] = (lax.shift_right_logical(cval, 7) * (B * 128)
                        + bi * 128 + jnp.bitwise_and(cval, 127))

        labj = lab_ref[:] != 0
        eqv = cval[:, :, None] == cval[:, None, :]
        jj = lax.broadcasted_iota(jnp.int32, (C, C), 0)
        jp = lax.broadcasted_iota(jnp.int32, (C, C), 1)
        earlier = (jp < jj)[None, :, :]
        dup = jnp.any(eqv & earlier & labj[:, None, :], axis=-1)
        cg = lax.shift_right_logical(cval, 6)
        selm = jnp.zeros((BB, C), jnp.bool_)
        for k in range(K):
            selm = jnp.logical_or(selm, cg == idx_cols[k])
        ks_ref[:] = (labj & jnp.logical_not(dup) & selm).astype(jnp.float32)

        zz = jnp.dot(y, et_ref[:], preferred_element_type=jnp.float32)
        for j in range(NZ):
            z_ref[j] = zz[:, j * 128:(j + 1) * 128]

        t = gl_ref[:].astype(jnp.float32)
        part = jnp.sum(jnp.maximum(y, 0.0) - y * t
                       + jnp.log1p(jnp.exp(-jnp.abs(y))))
        prev = jnp.where(ib == 0, 0.0, s2_ref[0, 0])
        s2_ref[0, 0] = prev + part


_NW = 32
_IDX_ROWS = (B * KC) // 128
_RPW = _IDX_ROWS // _NW
_CPW = B // _NW


@functools.partial(
    pl.kernel,
    out_type=(jax.ShapeDtypeStruct((B, KC), jnp.float32),
              jax.ShapeDtypeStruct((B, C), jnp.float32)),
    mesh=plsc.VectorSubcoreMesh(core_axis_name="c", subcore_axis_name="s"),
    scratch_types=[
        pltpu.VMEM((_RPW, 128), jnp.int32),
        pltpu.VMEM((_RPW, 128), jnp.float32),
        pltpu.VMEM((_CPW, C), jnp.int32),
        pltpu.VMEM((_CPW, C), jnp.float32),
        pltpu.SemaphoreType.DMA,
    ],
)
def _sc_gather(zflat_hbm, fidx_hbm, cfidx_hbm, out_hbm, zc_hbm,
               idx_v, out_v, cidx_v, zc_v, sem):
    wid = lax.axis_index("s") * 2 + lax.axis_index("c")
    base = wid * _RPW
    q = wid // (B // 128)
    b0 = (wid % (B // 128)) * 128
    bz0 = wid * _CPW
    pltpu.sync_copy(fidx_hbm.at[pl.ds(base, _RPW)], idx_v)
    pltpu.sync_copy(cfidx_hbm.at[pl.ds(bz0, _CPW)], cidx_v)
    cps = [pltpu.async_copy(zflat_hbm.at[idx_v.at[j]], out_v.at[j], sem)
           for j in range(_RPW)]
    cps += [pltpu.async_copy(zflat_hbm.at[cidx_v.at[i]], zc_v.at[i], sem)
            for i in range(_CPW)]
    for cp in cps:
        cp.wait()
    pltpu.sync_copy(out_v, out_hbm.at[pl.ds(b0, 128), pl.ds(q * 128, 128)])
    pltpu.sync_copy(zc_v, zc_hbm.at[pl.ds(bz0, _CPW)])


def _loss_body(lg_ref, zc_ref, ks_ref, s2_ref, out_ref):
    l = lg_ref[:]
    s1 = jnp.sum(jnp.maximum(l, 0.0) + jnp.log1p(jnp.exp(-jnp.abs(l))))
    s1 = s1 - jnp.sum(zc_ref[:] * ks_ref[:])
    out_ref[0, 0] = s1 / (B * KC) + s2_ref[0, 0] / (B * G)


def kernel(x, candidates, labels, group_labels, W1, b1, Wm, bm, Etab, group_y):
    x_tail = lax.slice(x, (0, ND * DK), (B, D))
    w1_tail = lax.slice(W1, (ND * DK, 0), (D, H))
    b1r = b1.reshape(1, H)
    bmr = bm.reshape(1, G)
    etT = Etab.T

    z3, fidx4, cfidx, keepsel, s2 = pl.pallas_call(
        _main_body,
        grid=(ND, NB),
        in_specs=[
            pl.BlockSpec((BB, DK), lambda idd, ib: (ib, idd)),
            pl.BlockSpec((BB, DT), lambda idd, ib: (ib, 0)),
            pl.BlockSpec((DK, H), lambda idd, ib: (idd, 0)),
            pl.BlockSpec((DT, H), lambda idd, ib: (0, 0)),
            pl.BlockSpec((1, H), lambda idd, ib: (0, 0)),
            pl.BlockSpec((H, G), lambda idd, ib: (0, 0)),
            pl.BlockSpec((1, G), lambda idd, ib: (0, 0)),
            pl.BlockSpec((G, L), lambda idd, ib: (0, 0)),
            pl.BlockSpec((BB, C), lambda idd, ib: (ib, 0)),
            pl.BlockSpec((BB, C), lambda idd, ib: (ib, 0)),
            pl.BlockSpec((BB, G), lambda idd, ib: (ib, 0)),
        ],
        out_specs=[
            pl.BlockSpec((NZ, BB, 128), lambda idd, ib: (0, ib, 0)),
            pl.BlockSpec((NQ, BB, 128), lambda idd, ib: (0, ib, 0)),
            pl.BlockSpec((BB, C), lambda idd, ib: (ib, 0)),
            pl.BlockSpec((BB, C), lambda idd, ib: (ib, 0)),
            pl.BlockSpec(memory_space=pltpu.SMEM),
        ],
        out_shape=[
            jax.ShapeDtypeStruct((NZ, B, 128), jnp.float32),
            jax.ShapeDtypeStruct((NQ, B, 128), jnp.int32),
            jax.ShapeDtypeStruct((B, C), jnp.int32),
            jax.ShapeDtypeStruct((B, C), jnp.float32),
            jax.ShapeDtypeStruct((1, 1), jnp.float32),
        ],
        scratch_shapes=[pltpu.VMEM((B, H), jnp.float32)],
    )(x, x_tail, W1, w1_tail, b1r, Wm, bmr, etT, candidates, labels,
      group_labels)

    zflat = z3.reshape(NZ * B * 128)
    fidx = fidx4.reshape(_IDX_ROWS, 128)
    logits, zc = _sc_gather(zflat, fidx, cfidx)

    loss = pl.pallas_call(
        _loss_body,
        in_specs=[
            pl.BlockSpec((B, KC), lambda: (0, 0)),
            pl.BlockSpec((B, C), lambda: (0, 0)),
            pl.BlockSpec((B, C), lambda: (0, 0)),
            pl.BlockSpec(memory_space=pltpu.SMEM),
        ],
        out_specs=pl.BlockSpec(memory_space=pltpu.SMEM),
        out_shape=jax.ShapeDtypeStruct((1, 1), jnp.float32),
    )(logits, zc, keepsel, s2)

    return logits, loss.reshape(())


# speedup vs baseline: 13.5163x; 1.0744x over previous
import functools

import jax
import jax.numpy as jnp
from jax import lax
from jax.experimental import pallas as pl
from jax.experimental.pallas import tpu as pltpu
from jax.experimental.pallas import tpu_sc as plsc

B, D, H, G, K, L, C = 1024, 5000, 1000, 64, 8, 4096, 32
GS = L // G
KC = K * GS
BB = 128
NB = B // BB
DK = 1664
ND = 3
DT = D - ND * DK


def _main_body(x_ref, xt_ref, w1_ref, w1t_ref, b1_ref, wm_ref, bm_ref,
               et_ref, cnd_ref, gl_ref, z_ref, ridx_ref, cfidx_ref, s2_ref,
               acc_ref):
    idd = pl.program_id(0)
    ib = pl.program_id(1)
    rows = pl.ds(ib * BB, BB)

    @pl.when(idd == 0)
    def _():
        acc_ref[rows, :] = jnp.zeros((BB, H), jnp.float32)

    acc_ref[rows, :] += jnp.dot(x_ref[:], w1_ref[:],
                                preferred_element_type=jnp.float32)

    @pl.when(idd == ND - 1)
    def _():
        h = jnp.maximum(
            acc_ref[rows, :]
            + jnp.dot(xt_ref[:], w1t_ref[:], preferred_element_type=jnp.float32)
            + b1_ref[:], 0.0)
        glog = jnp.dot(h, wm_ref[:], preferred_element_type=jnp.float32) + bm_ref[:]
        y = 1.0 / (1.0 + jnp.exp(-glog))

        ji = lax.broadcasted_iota(jnp.int32, (BB, G), 1)
        work = y
        idx_cols = []
        for _ in range(K):
            m = jnp.max(work, axis=1, keepdims=True)
            sel = jnp.min(jnp.where(work == m, ji, G), axis=1, keepdims=True)
            idx_cols.append(sel)
            work = jnp.where(ji == sel, -1.0, work)

        bi = lax.broadcasted_iota(jnp.int32, (BB, 1), 0) + ib * BB

        ki = lax.broadcasted_iota(jnp.int32, (BB, 2 * K), 1)
        r16 = jnp.broadcast_to(bi, (BB, 2 * K))
        for k in range(K):
            r16 = jnp.where(ki == k, idx_cols[k] * B + bi, r16)
        ridx_ref[:] = r16

        cval = cnd_ref[:]
        cfidx_ref[:] = (lax.shift_right_logical(cval, 6) * (B * 128)
                        + bi * 128 + jnp.bitwise_and(cval, GS - 1))

        zz = jnp.dot(y, et_ref[:], preferred_element_type=jnp.float32)
        for g in range(G):
            blk = zz[:, g * GS:(g + 1) * GS]
            z_ref[g] = jnp.concatenate([blk, blk], axis=1)

        t = gl_ref[:].astype(jnp.float32)
        part = jnp.sum(jnp.maximum(y, 0.0) - y * t
                       + jnp.log1p(jnp.exp(-jnp.abs(y))))
        prev = jnp.where(ib == 0, 0.0, s2_ref[0, 0])
        s2_ref[0, 0] = prev + part


_NW = 32
_BPW = B // _NW


@functools.partial(
    pl.kernel,
    out_type=jax.ShapeDtypeStruct((B, KC), jnp.float32),
    mesh=plsc.VectorSubcoreMesh(core_axis_name="c", subcore_axis_name="s"),
    scratch_types=[
        pltpu.VMEM((_BPW, 2 * K), jnp.int32),
        pltpu.VMEM((_BPW, 2 * K, 128), jnp.float32),
        pltpu.VMEM((_BPW, KC), jnp.float32),
        pltpu.SemaphoreType.DMA,
    ],
)
def _sc_gather(zrows_hbm, ridx_hbm, out_hbm, idx_v, rows_v, out_v, sem):
    wid = lax.axis_index("s") * 2 + lax.axis_index("c")
    b0 = wid * _BPW
    pltpu.sync_copy(ridx_hbm.at[pl.ds(b0, _BPW)], idx_v)
    cps = [pltpu.async_copy(zrows_hbm.at[idx_v.at[i]], rows_v.at[i], sem)
           for i in range(_BPW)]
    for cp in cps:
        cp.wait()

    def repack(i, carry):
        for k in range(K):
            for c in range(GS // 16):
                out_v[i, pl.ds(k * GS + c * 16, 16)] = (
                    rows_v[i, k, pl.ds(c * 16, 16)])
        return carry

    lax.fori_loop(0, _BPW, repack, 0)
    pltpu.sync_copy(out_v, out_hbm.at[pl.ds(b0, _BPW)])


@functools.partial(
    pl.kernel,
    out_type=jax.ShapeDtypeStruct((B, C), jnp.float32),
    mesh=plsc.VectorSubcoreMesh(core_axis_name="c", subcore_axis_name="s"),
    scratch_types=[
        pltpu.VMEM((_BPW, C), jnp.int32),
        pltpu.VMEM((_BPW, C), jnp.float32),
        pltpu.SemaphoreType.DMA,
    ],
)
def _sc_zc(zflat_hbm, cfidx_hbm, zc_hbm, cf_v, zc_v, sem):
    wid = lax.axis_index("s") * 2 + lax.axis_index("c")
    b0 = wid * _BPW
    pltpu.sync_copy(cfidx_hbm.at[pl.ds(b0, _BPW)], cf_v)
    cps = [pltpu.async_copy(zflat_hbm.at[cf_v.at[i]], zc_v.at[i], sem)
           for i in range(_BPW)]
    for cp in cps:
        cp.wait()
    pltpu.sync_copy(zc_v, zc_hbm.at[pl.ds(b0, _BPW)])


def _loss_body(lg_ref, grp_ref, cnd_ref, lab_ref, zc_ref, s2_ref, out_ref):
    l = lg_ref[:]
    cval = cnd_ref[:]
    labj = lab_ref[:] != 0
    ji = lax.broadcasted_iota(jnp.int32, (C, B), 0)
    dup = jnp.zeros((C, B), jnp.bool_)
    for jp in range(C - 1):
        dup = jnp.logical_or(
            dup, (cval == cval[jp:jp + 1, :]) & labj[jp:jp + 1, :] & (ji > jp))
    cg = lax.shift_right_logical(cval, 6)
    selm = jnp.zeros((C, B), jnp.bool_)
    for k in range(K):
        selm = jnp.logical_or(selm, cg == grp_ref[k:k + 1, :])
    ks = (labj & jnp.logical_not(dup) & selm).astype(jnp.float32)
    s1 = jnp.sum(jnp.maximum(l, 0.0) + jnp.log1p(jnp.exp(-jnp.abs(l))))
    s1 = s1 - jnp.sum(zc_ref[:] * ks)
    out_ref[0, 0] = s1 / (B * KC) + s2_ref[0, 0] / (B * G)


def kernel(x, candidates, labels, group_labels, W1, b1, Wm, bm, Etab, group_y):
    x_tail = lax.slice(x, (0, ND * DK), (B, D))
    w1_tail = lax.slice(W1, (ND * DK, 0), (D, H))
    b1r = b1.reshape(1, H)
    bmr = bm.reshape(1, G)
    etT = Etab.T

    zd, ridx, cfidx, s2 = pl.pallas_call(
        _main_body,
        grid=(ND, NB),
        in_specs=[
            pl.BlockSpec((BB, DK), lambda idd, ib: (ib, idd)),
            pl.BlockSpec((BB, DT), lambda idd, ib: (ib, 0)),
            pl.BlockSpec((DK, H), lambda idd, ib: (idd, 0)),
            pl.BlockSpec((DT, H), lambda idd, ib: (0, 0)),
            pl.BlockSpec((1, H), lambda idd, ib: (0, 0)),
            pl.BlockSpec((H, G), lambda idd, ib: (0, 0)),
            pl.BlockSpec((1, G), lambda idd, ib: (0, 0)),
            pl.BlockSpec((G, L), lambda idd, ib: (0, 0)),
            pl.BlockSpec((BB, C), lambda idd, ib: (ib, 0)),
            pl.BlockSpec((BB, G), lambda idd, ib: (ib, 0)),
        ],
        out_specs=[
            pl.BlockSpec((G, BB, 128), lambda idd, ib: (0, ib, 0)),
            pl.BlockSpec((BB, 2 * K), lambda idd, ib: (ib, 0)),
            pl.BlockSpec((BB, C), lambda idd, ib: (ib, 0)),
            pl.BlockSpec(memory_space=pltpu.SMEM),
        ],
        out_shape=[
            jax.ShapeDtypeStruct((G, B, 128), jnp.float32),
            jax.ShapeDtypeStruct((B, 2 * K), jnp.int32),
            jax.ShapeDtypeStruct((B, C), jnp.int32),
            jax.ShapeDtypeStruct((1, 1), jnp.float32),
        ],
        scratch_shapes=[pltpu.VMEM((B, H), jnp.float32)],
    )(x, x_tail, W1, w1_tail, b1r, Wm, bmr, etT, candidates, group_labels)

    zrows = zd.reshape(G * B, 128)
    logits = _sc_gather(zrows, ridx)
    zflat = zd.reshape(G * B * 128)
    zc = _sc_zc(zflat, cfidx)

    grpT = lax.shift_right_logical(ridx[:, :K], 10).T
    candT = candidates.T
    labT = labels.T
    zcT = zc.T

    loss = pl.pallas_call(
        _loss_body,
        in_specs=[
            pl.BlockSpec((B, KC), lambda: (0, 0)),
            pl.BlockSpec((K, B), lambda: (0, 0)),
            pl.BlockSpec((C, B), lambda: (0, 0)),
            pl.BlockSpec((C, B), lambda: (0, 0)),
            pl.BlockSpec((C, B), lambda: (0, 0)),
            pl.BlockSpec(memory_space=pltpu.SMEM),
        ],
        out_specs=pl.BlockSpec(memory_space=pltpu.SMEM),
        out_shape=jax.ShapeDtypeStruct((1, 1), jnp.float32),
    )(logits, grpT, candT, labT, zcT, s2)

    return logits, loss.reshape(())
